# baseline (device time: 104854 ns/iter reference)
import jax
import jax.numpy as jnp
from jax import lax
from jax.experimental import pallas as pl
from jax.experimental.pallas import tpu as pltpu

N_DEV = 8
N_PIECE = 1


def kernel(x, w_mat, scale_x, scale_w):
    m_per, k = x.shape
    _, n_per = w_mat.shape
    half = m_per // 2
    piece = half // N_PIECE

    s = (scale_x.astype(jnp.float32) * scale_w.astype(jnp.float32)).reshape(1, 1)

    def body(x_ref, w_ref, s_ref, out_hbm, xs_ref, w8_ref, out_buf,
             send_sems, recv_sems, copy_sems):
        my = lax.axis_index("i")
        left = lax.rem(my + N_DEV - 1, N_DEV)
        right = lax.rem(my + 1, N_DEV)

        barrier_sem = pltpu.get_barrier_semaphore()
        for nbr in (left, right):
            pl.semaphore_signal(
                barrier_sem, inc=1,
                device_id=(nbr,), device_id_type=pl.DeviceIdType.MESH,
            )
        pl.semaphore_wait(barrier_sem, 2)

        sc = s_ref[0, 0]

        dst = (right, left)

        def rows(d, p):
            return pl.ds(d * half + p * piece, piece)

        def start_send(d, h, p, c):
            rdma = pltpu.make_async_remote_copy(
                src_ref=xs_ref.at[c, rows(d, p), :],
                dst_ref=xs_ref.at[c, rows(d, p), :],
                send_sem=send_sems.at[d, h, p],
                recv_sem=recv_sems.at[d, h, p],
                device_id=(dst[d],),
                device_id_type=pl.DeviceIdType.MESH,
            )
            rdma.start()
            return rdma

        copies = []

        def compute_piece(c, d, p, slot):
            a = xs_ref[c, rows(d, p), :]
            y = jnp.dot(a, w8_ref[:, :], preferred_element_type=jnp.float32) * sc
            out_buf[c, rows(d, p), :] = y * jax.nn.sigmoid(y)
            cp = pltpu.make_async_copy(
                out_buf.at[c, rows(d, p), :],
                out_hbm.at[pl.ds(c * m_per + d * half + p * piece, piece), :],
                copy_sems.at[slot, d, p],
            )
            cp.start()
            copies.append(cp)

        sends = {}
        for p in range(N_PIECE):
            for d in range(2):
                xs_ref[my, rows(d, p), :] = (
                    x_ref[rows(d, p), :].astype(jnp.float8_e4m3fn))
                sends[(d, 0, p)] = start_send(d, 0, p, my)
        w8_ref[:, :] = w_ref[:, :].astype(jnp.float8_e5m2)
        for p in range(N_PIECE):
            for d in range(2):
                compute_piece(my, d, p, 0)

        for h in range(N_DEV - 1):
            rc = (lax.rem(my + N_DEV - h - 1, N_DEV),
                  lax.rem(my + h + 1, N_DEV))
            for p in range(N_PIECE):
                for d in range(2):
                    sends[(d, h, p)].wait_recv()
                    if h < N_DEV - 2:
                        sends[(d, h + 1, p)] = start_send(d, h + 1, p, rc[d])
                for d in range(2):
                    compute_piece(rc[d], d, p, h + 1)

        for rdma in sends.values():
            rdma.wait_send()
        for cp in copies:
            cp.wait()

    return pl.pallas_call(
        body,
        out_shape=jax.ShapeDtypeStruct((N_DEV * m_per, n_per), jnp.float32),
        in_specs=[
            pl.BlockSpec(memory_space=pltpu.VMEM),
            pl.BlockSpec(memory_space=pltpu.VMEM),
            pl.BlockSpec(memory_space=pltpu.SMEM),
        ],
        out_specs=pl.BlockSpec(memory_space=pl.ANY),
        scratch_shapes=[
            pltpu.VMEM((N_DEV, m_per, k), jnp.float8_e4m3fn),
            pltpu.VMEM((k, n_per), jnp.float8_e5m2),
            pltpu.VMEM((N_DEV, m_per, n_per), jnp.float32),
            pltpu.SemaphoreType.DMA((2, N_DEV - 1, N_PIECE)),
            pltpu.SemaphoreType.DMA((2, N_DEV - 1, N_PIECE)),
            pltpu.SemaphoreType.DMA((N_DEV, 2, N_PIECE)),
        ],
        compiler_params=pltpu.CompilerParams(collective_id=0),
    )(x, w_mat, s)


# device time: 95205 ns/iter; 1.1013x vs baseline; 1.1013x over previous
import jax
import jax.numpy as jnp
from jax import lax
from jax.experimental import pallas as pl
from jax.experimental.pallas import tpu as pltpu

N_DEV = 8
N_PIECE = 4


def kernel(x, w_mat, scale_x, scale_w):
    m_per, k = x.shape
    _, n_per = w_mat.shape
    half = m_per // 2
    piece = half // N_PIECE

    s = (scale_x.astype(jnp.float32) * scale_w.astype(jnp.float32)).reshape(1, 1)

    def body(x_ref, w_ref, s_ref, out_hbm, xs_ref, w8_ref, out_buf,
             send_sems, recv_sems, copy_sems):
        my = lax.axis_index("i")
        left = lax.rem(my + N_DEV - 1, N_DEV)
        right = lax.rem(my + 1, N_DEV)

        barrier_sem = pltpu.get_barrier_semaphore()
        for nbr in (left, right):
            pl.semaphore_signal(
                barrier_sem, inc=1,
                device_id=(nbr,), device_id_type=pl.DeviceIdType.MESH,
            )
        pl.semaphore_wait(barrier_sem, 2)

        sc = s_ref[0, 0]

        dst = (right, left)

        def rows(d, p):
            return pl.ds(d * half + p * piece, piece)

        def start_send(d, h, p, c):
            rdma = pltpu.make_async_remote_copy(
                src_ref=xs_ref.at[c, rows(d, p), :],
                dst_ref=xs_ref.at[c, rows(d, p), :],
                send_sem=send_sems.at[d, h, p],
                recv_sem=recv_sems.at[d, h, p],
                device_id=(dst[d],),
                device_id_type=pl.DeviceIdType.MESH,
            )
            rdma.start()
            return rdma

        copies = []

        def compute_piece(c, d, p, slot):
            a = xs_ref[c, rows(d, p), :]
            y = jnp.dot(a, w8_ref[:, :], preferred_element_type=jnp.float32) * sc
            out_buf[c, rows(d, p), :] = y * jax.nn.sigmoid(y)
            cp = pltpu.make_async_copy(
                out_buf.at[c, rows(d, p), :],
                out_hbm.at[pl.ds(c * m_per + d * half + p * piece, piece), :],
                copy_sems.at[slot, d, p],
            )
            cp.start()
            copies.append(cp)

        sends = {}
        for p in range(N_PIECE):
            for d in range(2):
                xs_ref[my, rows(d, p), :] = (
                    x_ref[rows(d, p), :].astype(jnp.float8_e4m3fn))
                sends[(d, 0, p)] = start_send(d, 0, p, my)
        w8_ref[:, :] = w_ref[:, :].astype(jnp.float8_e5m2)
        for p in range(N_PIECE):
            for d in range(2):
                compute_piece(my, d, p, 0)

        for h in range(N_DEV - 1):
            rc = (lax.rem(my + N_DEV - h - 1, N_DEV),
                  lax.rem(my + h + 1, N_DEV))
            for p in range(N_PIECE):
                for d in range(2):
                    sends[(d, h, p)].wait_recv()
                    if h < N_DEV - 2:
                        sends[(d, h + 1, p)] = start_send(d, h + 1, p, rc[d])
                for d in range(2):
                    compute_piece(rc[d], d, p, h + 1)

        for rdma in sends.values():
            rdma.wait_send()
        for cp in copies:
            cp.wait()

    return pl.pallas_call(
        body,
        out_shape=jax.ShapeDtypeStruct((N_DEV * m_per, n_per), jnp.float32),
        in_specs=[
            pl.BlockSpec(memory_space=pltpu.VMEM),
            pl.BlockSpec(memory_space=pltpu.VMEM),
            pl.BlockSpec(memory_space=pltpu.SMEM),
        ],
        out_specs=pl.BlockSpec(memory_space=pl.ANY),
        scratch_shapes=[
            pltpu.VMEM((N_DEV, m_per, k), jnp.float8_e4m3fn),
            pltpu.VMEM((k, n_per), jnp.float8_e5m2),
            pltpu.VMEM((N_DEV, m_per, n_per), jnp.float32),
            pltpu.SemaphoreType.DMA((2, N_DEV - 1, N_PIECE)),
            pltpu.SemaphoreType.DMA((2, N_DEV - 1, N_PIECE)),
            pltpu.SemaphoreType.DMA((N_DEV, 2, N_PIECE)),
        ],
        compiler_params=pltpu.CompilerParams(collective_id=0),
    )(x, w_mat, s)


# device time: 90269 ns/iter; 1.1616x vs baseline; 1.0547x over previous
import jax
import jax.numpy as jnp
from jax import lax
from jax.experimental import pallas as pl
from jax.experimental.pallas import tpu as pltpu

N_DEV = 8
N_Q = 4
PIECE = 128


def kernel(x, w_mat, scale_x, scale_w):
    m_per, k = x.shape
    _, n_per = w_mat.shape

    s = (scale_x.astype(jnp.float32) * scale_w.astype(jnp.float32)).reshape(1, 1)

    def body(x_ref, w_ref, s_ref, out_hbm, xs_ref, w8_ref, out_buf,
             z_send_sems, z_recv_sems, send_sems, recv_sems, copy_sems):
        my = lax.axis_index("i")
        j = lax.rem(my, 4)
        p4 = my - j
        right = p4 + lax.rem(j + 1, 4)
        left = p4 + lax.rem(j + 3, 4)
        zp = lax.rem(my + 4, N_DEV)

        barrier_sem = pltpu.get_barrier_semaphore()
        for nbr in (left, right, zp):
            pl.semaphore_signal(
                barrier_sem, inc=1,
                device_id=(nbr,), device_id_type=pl.DeviceIdType.MESH,
            )
        pl.semaphore_wait(barrier_sem, 3)

        sc = s_ref[0, 0]

        def qrows(q):
            return pl.ds(q * PIECE, PIECE)

        def ring_send(d, t, kk, c):
            q = 2 * d + (kk % 2)
            rdma = pltpu.make_async_remote_copy(
                src_ref=xs_ref.at[c, qrows(q), :],
                dst_ref=xs_ref.at[c, qrows(q), :],
                send_sem=send_sems.at[d, t, kk],
                recv_sem=recv_sems.at[d, t, kk],
                device_id=(right if d == 0 else left,),
                device_id_type=pl.DeviceIdType.MESH,
            )
            rdma.start()
            return rdma

        copies = []

        def compute_piece(c, q):
            a = xs_ref[c, qrows(q), :]
            y = jnp.dot(a, w8_ref[:, :], preferred_element_type=jnp.float32) * sc
            out_buf[c, qrows(q), :] = y * jax.nn.sigmoid(y)
            cp = pltpu.make_async_copy(
                out_buf.at[c, qrows(q), :],
                out_hbm.at[pl.ds(c * m_per + q * PIECE, PIECE), :],
                copy_sems.at[len(copies)],
            )
            cp.start()
            copies.append(cp)

        sends = {}
        z_rdmas = []
        z_order = ((0, 0), (2, 1), (1, 0), (3, 1))
        for q, d in z_order:
            xs_ref[my, qrows(q), :] = x_ref[qrows(q), :].astype(jnp.float8_e4m3fn)
            z = pltpu.make_async_remote_copy(
                src_ref=xs_ref.at[my, qrows(q), :],
                dst_ref=xs_ref.at[my, qrows(q), :],
                send_sem=z_send_sems.at[q],
                recv_sem=z_recv_sems.at[q],
                device_id=(zp,), device_id_type=pl.DeviceIdType.MESH,
            )
            z.start()
            z_rdmas.append(z)
            sends[(d, 0, q % 2)] = ring_send(d, 0, q % 2, my)
        w8_ref[:, :] = w_ref[:, :].astype(jnp.float8_e5m2)
        for q in range(N_Q):
            compute_piece(my, q)

        for z, (q, d) in zip(z_rdmas, z_order):
            z.wait_recv()
            sends[(d, 0, 2 + q % 2)] = ring_send(d, 0, 2 + q % 2, zp)
            compute_piece(zp, q)

        for t in (1, 2):
            m = (lax.rem(j - t + 4, 4), lax.rem(j + t, 4))
            for kk in range(4):
                for d in (0, 1):
                    cu = p4 + m[d]
                    c = cu if kk < 2 else lax.rem(cu + 4, N_DEV)
                    sends[(d, t - 1, kk)].wait_recv()
                    sends[(d, t, kk)] = ring_send(d, t, kk, c)
                for d in (0, 1):
                    cu = p4 + m[d]
                    c = cu if kk < 2 else lax.rem(cu + 4, N_DEV)
                    compute_piece(c, 2 * d + (kk % 2))

        m = (lax.rem(j + 1, 4), lax.rem(j + 3, 4))
        for kk in range(4):
            for d in (0, 1):
                sends[(d, 2, kk)].wait_recv()
            for d in (0, 1):
                cu = p4 + m[d]
                c = cu if kk < 2 else lax.rem(cu + 4, N_DEV)
                compute_piece(c, 2 * d + (kk % 2))

        for z in z_rdmas:
            z.wait_send()
        for rdma in sends.values():
            rdma.wait_send()
        for cp in copies:
            cp.wait()

    return pl.pallas_call(
        body,
        out_shape=jax.ShapeDtypeStruct((N_DEV * m_per, n_per), jnp.float32),
        in_specs=[
            pl.BlockSpec(memory_space=pltpu.VMEM),
            pl.BlockSpec(memory_space=pltpu.VMEM),
            pl.BlockSpec(memory_space=pltpu.SMEM),
        ],
        out_specs=pl.BlockSpec(memory_space=pl.ANY),
        scratch_shapes=[
            pltpu.VMEM((N_DEV, m_per, k), jnp.float8_e4m3fn),
            pltpu.VMEM((k, n_per), jnp.float8_e5m2),
            pltpu.VMEM((N_DEV, m_per, n_per), jnp.float32),
            pltpu.SemaphoreType.DMA((N_Q,)),
            pltpu.SemaphoreType.DMA((N_Q,)),
            pltpu.SemaphoreType.DMA((2, 3, 4)),
            pltpu.SemaphoreType.DMA((2, 3, 4)),
            pltpu.SemaphoreType.DMA((32,)),
        ],
        compiler_params=pltpu.CompilerParams(collective_id=0),
    )(x, w_mat, s)


# device time: 84821 ns/iter; 1.2362x vs baseline; 1.0642x over previous
import jax
import jax.numpy as jnp
from jax import lax
from jax.experimental import pallas as pl
from jax.experimental.pallas import tpu as pltpu

N_DEV = 8


def kernel(x, w_mat, scale_x, scale_w):
    m_per, k = x.shape
    _, n_per = w_mat.shape

    s = (scale_x.astype(jnp.float32) * scale_w.astype(jnp.float32)).reshape(1, 1)

    def body(x_hbm, w_hbm, s_ref, out_hbm, xs_ref, xf_ref, wf_ref, w8_ref,
             out_buf, x_sems, w_sem, z_send_sems, z_recv_sems,
             send_sems, recv_sems, copy_sems):
        my = lax.axis_index("i")
        j = lax.rem(my, 4)
        p4 = my - j
        right = p4 + lax.rem(j + 1, 4)
        left = p4 + lax.rem(j + 3, 4)
        zp = lax.rem(my + 4, N_DEV)

        x_dmas = {}
        for q in (0, 2, 1, 3):
            cp = pltpu.make_async_copy(
                x_hbm.at[pl.ds(q * 128, 128), :],
                xf_ref.at[pl.ds(q * 128, 128), :],
                x_sems.at[q],
            )
            cp.start()
            x_dmas[q] = cp
        w_dma = pltpu.make_async_copy(w_hbm, wf_ref, w_sem.at[0])
        w_dma.start()

        barrier_sem = pltpu.get_barrier_semaphore()
        for nbr in (left, right, zp):
            pl.semaphore_signal(
                barrier_sem, inc=1,
                device_id=(nbr,), device_id_type=pl.DeviceIdType.MESH,
            )
        pl.semaphore_wait(barrier_sem, 3)

        sc = s_ref[0, 0]

        def ring_send(d, t, kk, c):
            if kk < 2:
                off, nr = d * 256 + kk * 128, 128
            else:
                off, nr = d * 256 + (kk - 2) * 64, 64
            rdma = pltpu.make_async_remote_copy(
                src_ref=xs_ref.at[c, pl.ds(off, nr), :],
                dst_ref=xs_ref.at[c, pl.ds(off, nr), :],
                send_sem=send_sems.at[d, t, kk],
                recv_sem=recv_sems.at[d, t, kk],
                device_id=(right if d == 0 else left,),
                device_id_type=pl.DeviceIdType.MESH,
            )
            rdma.start()
            return rdma

        copies = []

        def compute_piece(c, off, nr):
            a = xs_ref[c, pl.ds(off, nr), :]
            y = jnp.dot(a, w8_ref[:, :], preferred_element_type=jnp.float32) * sc
            out_buf[c, pl.ds(off, nr), :] = y * jax.nn.sigmoid(y)
            cp = pltpu.make_async_copy(
                out_buf.at[c, pl.ds(off, nr), :],
                out_hbm.at[pl.ds(c * m_per + off, nr), :],
                copy_sems.at[len(copies)],
            )
            cp.start()
            copies.append(cp)

        sends = {}
        z_rdmas = []

        def z_send(d, i):
            off = d * 256 + i * 64
            z = pltpu.make_async_remote_copy(
                src_ref=xs_ref.at[my, pl.ds(off, 64), :],
                dst_ref=xs_ref.at[my, pl.ds(off, 64), :],
                send_sem=z_send_sems.at[d, i],
                recv_sem=z_recv_sems.at[d, i],
                device_id=(zp,), device_id_type=pl.DeviceIdType.MESH,
            )
            z.start()
            z_rdmas.append(z)
            return z

        z_waits = {}
        for q, d in ((0, 0), (2, 1), (1, 0), (3, 1)):
            x_dmas[q].wait()
            xs_ref[my, pl.ds(q * 128, 128), :] = (
                xf_ref[pl.ds(q * 128, 128), :].astype(jnp.float8_e4m3fn))
            p = q % 2
            sends[(d, 0, p)] = ring_send(d, 0, p, my)
            z_waits[(d, 2 * p)] = z_send(d, 2 * p)
            z_waits[(d, 2 * p + 1)] = z_send(d, 2 * p + 1)

        w_dma.wait()
        w8_ref[:, :] = wf_ref[:, :].astype(jnp.float8_e5m2)
        for q in range(4):
            compute_piece(my, q * 128, 128)

        for d, i in ((0, 0), (0, 1), (1, 0), (1, 1), (0, 2), (0, 3),
                     (1, 2), (1, 3)):
            z_waits[(d, i)].wait_recv()
            sends[(d, 0, 2 + i)] = ring_send(d, 0, 2 + i, zp)
            compute_piece(zp, d * 256 + i * 64, 64)

        def unit_chunk(d, t, kk):
            m = lax.rem(j - t + 4, 4) if d == 0 else lax.rem(j + t, 4)
            cu = p4 + m
            return cu if kk < 2 else lax.rem(cu + 4, N_DEV)

        def piece_off(d, kk):
            if kk < 2:
                return d * 256 + kk * 128, 128
            return d * 256 + (kk - 2) * 64, 64

        for t in (1, 2):
            for kk in range(6):
                for d in (0, 1):
                    sends[(d, t - 1, kk)].wait_recv()
                    sends[(d, t, kk)] = ring_send(d, t, kk, unit_chunk(d, t, kk))
                for d in (0, 1):
                    off, nr = piece_off(d, kk)
                    compute_piece(unit_chunk(d, t, kk), off, nr)

        for kk in range(6):
            for d in (0, 1):
                sends[(d, 2, kk)].wait_recv()
            for d in (0, 1):
                off, nr = piece_off(d, kk)
                compute_piece(unit_chunk(d, 3, kk), off, nr)

        for z in z_rdmas:
            z.wait_send()
        for rdma in sends.values():
            rdma.wait_send()
        for cp in copies:
            cp.wait()

    return pl.pallas_call(
        body,
        out_shape=jax.ShapeDtypeStruct((N_DEV * m_per, n_per), jnp.float32),
        in_specs=[
            pl.BlockSpec(memory_space=pl.ANY),
            pl.BlockSpec(memory_space=pl.ANY),
            pl.BlockSpec(memory_space=pltpu.SMEM),
        ],
        out_specs=pl.BlockSpec(memory_space=pl.ANY),
        scratch_shapes=[
            pltpu.VMEM((N_DEV, m_per, k), jnp.float8_e4m3fn),
            pltpu.VMEM((m_per, k), jnp.float32),
            pltpu.VMEM((k, n_per), jnp.float32),
            pltpu.VMEM((k, n_per), jnp.float8_e5m2),
            pltpu.VMEM((N_DEV, m_per, n_per), jnp.float32),
            pltpu.SemaphoreType.DMA((4,)),
            pltpu.SemaphoreType.DMA((1,)),
            pltpu.SemaphoreType.DMA((2, 4)),
            pltpu.SemaphoreType.DMA((2, 4)),
            pltpu.SemaphoreType.DMA((2, 3, 6)),
            pltpu.SemaphoreType.DMA((2, 3, 6)),
            pltpu.SemaphoreType.DMA((48,)),
        ],
        compiler_params=pltpu.CompilerParams(
            collective_id=0, vmem_limit_bytes=64 * 1024 * 1024,
        ),
    )(x, w_mat, s)


# device time: 82468 ns/iter; 1.2715x vs baseline; 1.0285x over previous
import jax
import jax.numpy as jnp
from jax import lax
from jax.experimental import pallas as pl
from jax.experimental.pallas import tpu as pltpu

N_DEV = 8


def kernel(x, w_mat, scale_x, scale_w):
    m_per, k = x.shape
    _, n_per = w_mat.shape

    s = (scale_x.astype(jnp.float32) * scale_w.astype(jnp.float32)).reshape(1, 1)

    def body(x_hbm, w_hbm, s_ref, out_hbm, xs_ref, xf_ref, wf_ref, w8_ref,
             out_buf, x_sems, w_sem, z_send_sems, z_recv_sems,
             send_sems, recv_sems, copy_sems):
        my = lax.axis_index("i")
        j = lax.rem(my, 4)
        p4 = my - j
        right = p4 + lax.rem(j + 1, 4)
        left = p4 + lax.rem(j + 3, 4)
        zp = lax.rem(my + 4, N_DEV)

        x_dmas = {}
        for q in (0, 2, 1, 3):
            cp = pltpu.make_async_copy(
                x_hbm.at[pl.ds(q * 128, 128), :],
                xf_ref.at[pl.ds(q * 128, 128), :],
                x_sems.at[q],
            )
            cp.start()
            x_dmas[q] = cp
        w_dma = pltpu.make_async_copy(w_hbm, wf_ref, w_sem.at[0])
        w_dma.start()

        barrier_sem = pltpu.get_barrier_semaphore()
        for nbr in (left, right, zp):
            pl.semaphore_signal(
                barrier_sem, inc=1,
                device_id=(nbr,), device_id_type=pl.DeviceIdType.MESH,
            )
        pl.semaphore_wait(barrier_sem, 3)

        sc = s_ref[0, 0]

        def ring_send(d, t, kk, c):
            if kk < 2:
                off, nr = d * 256 + kk * 128, 128
            else:
                off, nr = d * 256 + (kk - 2) * 64, 64
            rdma = pltpu.make_async_remote_copy(
                src_ref=xs_ref.at[c, pl.ds(off, nr), :],
                dst_ref=xs_ref.at[c, pl.ds(off, nr), :],
                send_sem=send_sems.at[d, t, kk],
                recv_sem=recv_sems.at[d, t, kk],
                device_id=(right if d == 0 else left,),
                device_id_type=pl.DeviceIdType.MESH,
            )
            rdma.start()
            return rdma

        copies = []

        def compute_piece(c, off, nr):
            a = xs_ref[c, pl.ds(off, nr), :]
            y = jnp.dot(a, w8_ref[:, :], preferred_element_type=jnp.float32) * sc
            out_buf[c, pl.ds(off, nr), :] = y * jax.nn.sigmoid(y)
            cp = pltpu.make_async_copy(
                out_buf.at[c, pl.ds(off, nr), :],
                out_hbm.at[pl.ds(c * m_per + off, nr), :],
                copy_sems.at[len(copies)],
            )
            cp.start()
            copies.append(cp)

        sends = {}
        z_rdmas = []

        def z_send(d, i):
            off = d * 256 + i * 64
            z = pltpu.make_async_remote_copy(
                src_ref=xs_ref.at[my, pl.ds(off, 64), :],
                dst_ref=xs_ref.at[my, pl.ds(off, 64), :],
                send_sem=z_send_sems.at[d, i],
                recv_sem=z_recv_sems.at[d, i],
                device_id=(zp,), device_id_type=pl.DeviceIdType.MESH,
            )
            z.start()
            z_rdmas.append(z)
            return z

        z_waits = {}
        for q, d in ((0, 0), (2, 1), (1, 0), (3, 1)):
            x_dmas[q].wait()
            xs_ref[my, pl.ds(q * 128, 128), :] = (
                xf_ref[pl.ds(q * 128, 128), :].astype(jnp.float8_e4m3fn))
            p = q % 2
            sends[(d, 0, p)] = ring_send(d, 0, p, my)
            z_waits[(d, 2 * p)] = z_send(d, 2 * p)
            z_waits[(d, 2 * p + 1)] = z_send(d, 2 * p + 1)

        w_dma.wait()
        w8_ref[:, :] = wf_ref[:, :].astype(jnp.float8_e5m2)
        for q in range(4):
            compute_piece(my, q * 128, 128)

        def unit_chunk(d, t, kk):
            m = lax.rem(j - t + 4, 4) if d == 0 else lax.rem(j + t, 4)
            cu = p4 + m
            return cu if kk < 2 else lax.rem(cu + 4, N_DEV)

        def piece_off(d, kk):
            if kk < 2:
                return d * 256 + kk * 128, 128
            return d * 256 + (kk - 2) * 64, 64

        def inject(d, i):
            z_waits[(d, i)].wait_recv()
            sends[(d, 0, 2 + i)] = ring_send(d, 0, 2 + i, zp)
            compute_piece(zp, d * 256 + i * 64, 64)

        def hop(t, kk):
            for d in (0, 1):
                sends[(d, t - 1, kk)].wait_recv()
                sends[(d, t, kk)] = ring_send(d, t, kk, unit_chunk(d, t, kk))
            for d in (0, 1):
                off, nr = piece_off(d, kk)
                compute_piece(unit_chunk(d, t, kk), off, nr)

        for d, i in ((0, 0), (0, 1), (1, 0), (1, 1), (0, 2), (0, 3)):
            inject(d, i)
        hop(1, 0)
        hop(1, 1)
        inject(1, 2)
        inject(1, 3)
        for kk in range(2, 6):
            hop(1, kk)
        for kk in range(6):
            hop(2, kk)

        for kk in range(6):
            for d in (0, 1):
                sends[(d, 2, kk)].wait_recv()
            for d in (0, 1):
                off, nr = piece_off(d, kk)
                compute_piece(unit_chunk(d, 3, kk), off, nr)

        for z in z_rdmas:
            z.wait_send()
        for rdma in sends.values():
            rdma.wait_send()
        for cp in copies:
            cp.wait()

    return pl.pallas_call(
        body,
        out_shape=jax.ShapeDtypeStruct((N_DEV * m_per, n_per), jnp.float32),
        in_specs=[
            pl.BlockSpec(memory_space=pl.ANY),
            pl.BlockSpec(memory_space=pl.ANY),
            pl.BlockSpec(memory_space=pltpu.SMEM),
        ],
        out_specs=pl.BlockSpec(memory_space=pl.ANY),
        scratch_shapes=[
            pltpu.VMEM((N_DEV, m_per, k), jnp.float8_e4m3fn),
            pltpu.VMEM((m_per, k), jnp.float32),
            pltpu.VMEM((k, n_per), jnp.float32),
            pltpu.VMEM((k, n_per), jnp.float8_e5m2),
            pltpu.VMEM((N_DEV, m_per, n_per), jnp.float32),
            pltpu.SemaphoreType.DMA((4,)),
            pltpu.SemaphoreType.DMA((1,)),
            pltpu.SemaphoreType.DMA((2, 4)),
            pltpu.SemaphoreType.DMA((2, 4)),
            pltpu.SemaphoreType.DMA((2, 3, 6)),
            pltpu.SemaphoreType.DMA((2, 3, 6)),
            pltpu.SemaphoreType.DMA((48,)),
        ],
        compiler_params=pltpu.CompilerParams(
            collective_id=0, vmem_limit_bytes=64 * 1024 * 1024,
        ),
    )(x, w_mat, s)
